# baseline (speedup 1.0000x reference)
import functools

import jax
import jax.numpy as jnp
from jax import lax
from jax.experimental import pallas as pl
from jax.experimental.pallas import tpu as pltpu
from jax.experimental.pallas import tpu_sc as plsc

N_NODES = 10000
N_EDGES = 160000
E_PAD = 163840
NC = 2
NS = 16
NW = NC * NS
EPT = E_PAD // NW
CHUNK = 128
NCHUNK = EPT // CHUNK
STAGE = 1024
NSTAGE = EPT // STAGE
ROWS_PER_TILE = N_NODES // NS

_mesh = plsc.VectorSubcoreMesh(core_axis_name="c", subcore_axis_name="s")


def _prep_body(g0, mask, Wp, bp, Wt, bt, Wr, br, x_out, root_out):
    prep = jnp.dot(g0[...], Wp[...], preferred_element_type=jnp.float32) + bp[...]
    x0 = jnp.maximum(jnp.where(mask[...] == 0, prep, 0.0), 0.0)
    x = jnp.maximum(
        jnp.dot(x0, Wt[...], preferred_element_type=jnp.float32) + bt[...], 0.0)
    x_out[...] = x
    root_out[...] = (
        jnp.dot(x, Wr[...], preferred_element_type=jnp.float32) + br[...])


def _edge_body(eaT, xT, WeT, beT, W2T, B2T, out):
    hT = jax.nn.sigmoid(
        jnp.dot(WeT[...], eaT[...], preferred_element_type=jnp.float32)
        + beT[...])
    x = xT[...]
    b = x.shape[1]
    zT = (jnp.broadcast_to(hT[:, None, :], (16, 16, b)).reshape(256, b)
          * jnp.broadcast_to(x[None, :, :], (16, 16, b)).reshape(256, b))
    msgT = (jnp.dot(W2T[...], zT, preferred_element_type=jnp.float32)
            + jnp.dot(B2T[...], x, preferred_element_type=jnp.float32))
    pad = jnp.concatenate(
        [jnp.ones((1, b), jnp.float32), jnp.zeros((7, b), jnp.float32)], axis=0)
    msg32 = jnp.concatenate([msgT, pad], axis=0)
    eid = (jax.lax.broadcasted_iota(jnp.int32, (1, b), 1)
           + pl.program_id(0) * b)
    live = (eid < N_EDGES).astype(jnp.float32)
    out[...] = msg32 * live


def _final_body(p, root, Wo, bo, y_out):
    s = p[0] + p[1]
    cnt = jnp.maximum(s[:, 24:25], 1.0)
    mean_m = s[:, :16] / cnt
    h = jnp.concatenate(
        [mean_m + root[:, :16], s[:, 16:24] + root[:, 16:24]], axis=1)
    h = jnp.maximum(h, 0.0)
    y_out[...] = jnp.dot(h, Wo[...], preferred_element_type=jnp.float32) + bo[...]


@functools.partial(
    pl.kernel,
    mesh=_mesh,
    out_type=jax.ShapeDtypeStruct((16, E_PAD), jnp.float32),
    scratch_types=[
        pltpu.VMEM((NCHUNK, CHUNK), jnp.int32),
        pltpu.VMEM((STAGE, 16), jnp.float32),
        pltpu.VMEM((16, EPT), jnp.float32),
        pltpu.SemaphoreType.DMA,
    ],
    compiler_params=pltpu.CompilerParams(use_tc_tiling_on_sc=False,
                                         needs_layout_passes=False),
)
def _sc_gather(src_idx_hbm, x_hbm, out_hbm, idx_v, rows_v, xT_v, sem):
    c = lax.axis_index("c")
    s = lax.axis_index("s")
    wid = c * NS + s
    pltpu.sync_copy(src_idx_hbm.at[wid], idx_v)
    lane = lax.iota(jnp.int32, 16)

    def stage_body(k, carry):
        def fetch(j, carry2):
            pltpu.async_copy(
                x_hbm.at[idx_v.at[k * (STAGE // CHUNK) + j]],
                rows_v.at[pl.ds(j * CHUNK, CHUNK)], sem).wait()
            return carry2

        lax.fori_loop(0, STAGE // CHUNK, fetch, 0)

        def transpose_group(g, carry3):
            e0 = g * 16
            row_idx = e0 + lane
            for i in range(16):
                v = plsc.load_gather(
                    rows_v, [row_idx, jnp.full((16,), i, jnp.int32)])
                xT_v[i, pl.ds(k * STAGE + e0, 16)] = v
            return carry3

        lax.fori_loop(0, STAGE // 16, transpose_group, 0)
        return carry

    lax.fori_loop(0, NSTAGE, stage_body, 0)
    pltpu.sync_copy(xT_v, out_hbm.at[:, pl.ds(wid * EPT, EPT)])


@functools.partial(
    pl.kernel,
    mesh=_mesh,
    out_type=jax.ShapeDtypeStruct((NC, N_NODES, 32), jnp.float32),
    scratch_types=[
        pltpu.VMEM((NCHUNK, CHUNK), jnp.int32),
        pltpu.VMEM((32, STAGE), jnp.float32),
        pltpu.VMEM((STAGE, 32), jnp.float32),
        pltpu.VMEM_SHARED((N_NODES, 32), jnp.float32),
    ],
    compiler_params=pltpu.CompilerParams(use_tc_tiling_on_sc=False,
                                         needs_layout_passes=False),
)
def _sc_scatter(dst_idx_hbm, msgT_hbm, zeros_hbm, out_hbm,
                idx_v, slab_v, msg_v, accum):
    c = lax.axis_index("c")
    s = lax.axis_index("s")
    wid = c * NS + s
    pltpu.sync_copy(zeros_hbm.at[pl.ds(s * ROWS_PER_TILE, ROWS_PER_TILE)],
                    accum.at[pl.ds(s * ROWS_PER_TILE, ROWS_PER_TILE)])
    pltpu.sync_copy(dst_idx_hbm.at[wid], idx_v)
    plsc.subcore_barrier()

    base = wid * EPT
    lane = lax.iota(jnp.int32, 16)

    def stage_body(k, carry):
        pltpu.sync_copy(msgT_hbm.at[:, pl.ds(base + k * STAGE, STAGE)], slab_v)

        def transpose_group(g, carry2):
            e0 = g * 16
            for q in range(16):
                e = e0 + q
                for o0 in (0, 16):
                    v = plsc.load_gather(
                        slab_v, [o0 + lane, jnp.full((16,), 0, jnp.int32) + e])
                    msg_v[e, pl.ds(o0, 16)] = v
            return carry2

        lax.fori_loop(0, STAGE // 16, transpose_group, 0)

        def scat(j, carry3):
            pltpu.sync_copy(msg_v.at[pl.ds(j * CHUNK, CHUNK)],
                            accum.at[idx_v.at[k * (STAGE // CHUNK) + j]],
                            add=True)
            return carry3

        lax.fori_loop(0, STAGE // CHUNK, scat, 0)
        return carry

    lax.fori_loop(0, NSTAGE, stage_body, 0)
    plsc.subcore_barrier()
    pltpu.sync_copy(accum.at[pl.ds(s * ROWS_PER_TILE, ROWS_PER_TILE)],
                    out_hbm.at[c, pl.ds(s * ROWS_PER_TILE, ROWS_PER_TILE)])


def kernel(group0, edge_index, group_mask, edge_attr,
           W_prep, b_prep, W_tube, b_tube,
           W_e1, b_e1,
           W_em, b_em, W_root_m, bias_m,
           W_ea, b_ea, W_root_a, bias_a,
           W_out, b_out):
    f32 = jnp.float32
    mask2d = group_mask.astype(jnp.int32).reshape(N_NODES, 1)
    epad = jnp.zeros((E_PAD - N_EDGES,), jnp.int32)
    src_idx = jnp.concatenate(
        [edge_index[0].astype(jnp.int32), epad]).reshape(NW, NCHUNK, CHUNK)
    dst_idx = jnp.concatenate(
        [edge_index[1].astype(jnp.int32), epad]).reshape(NW, NCHUNK, CHUNK)

    Wr = jnp.concatenate([W_root_m, W_root_a], axis=1)
    br = jnp.concatenate([bias_m, bias_a]).reshape(1, 24)
    W2 = jnp.concatenate([W_em.reshape(16, 16, 16).reshape(256, 16),
                          W_ea.reshape(16, 16, 8).reshape(256, 8)], axis=1)
    B2 = jnp.concatenate([b_em.reshape(16, 16), b_ea.reshape(16, 8)], axis=1)
    W2T = W2.T
    B2T = B2.T
    WeT = W_e1.T
    beT = b_e1.reshape(16, 1)
    eaT = jnp.pad(edge_attr.T, ((0, 0), (0, E_PAD - N_EDGES)))

    nblk = 2000
    full = lambda i: (0, 0)
    x, root = pl.pallas_call(
        _prep_body,
        grid=(N_NODES // nblk,),
        in_specs=[
            pl.BlockSpec((nblk, 128), lambda i: (i, 0)),
            pl.BlockSpec((nblk, 1), lambda i: (i, 0)),
            pl.BlockSpec((128, 64), full),
            pl.BlockSpec((1, 64), full),
            pl.BlockSpec((64, 16), full),
            pl.BlockSpec((1, 16), full),
            pl.BlockSpec((16, 24), full),
            pl.BlockSpec((1, 24), full),
        ],
        out_specs=[
            pl.BlockSpec((nblk, 16), lambda i: (i, 0)),
            pl.BlockSpec((nblk, 24), lambda i: (i, 0)),
        ],
        out_shape=[
            jax.ShapeDtypeStruct((N_NODES, 16), f32),
            jax.ShapeDtypeStruct((N_NODES, 24), f32),
        ],
    )(group0, mask2d, W_prep, b_prep.reshape(1, 64),
      W_tube, b_tube.reshape(1, 16), Wr, br)

    xT = _sc_gather(src_idx, x)

    eblk = 4096
    msgT = pl.pallas_call(
        _edge_body,
        grid=(E_PAD // eblk,),
        in_specs=[
            pl.BlockSpec((4, eblk), lambda i: (0, i)),
            pl.BlockSpec((16, eblk), lambda i: (0, i)),
            pl.BlockSpec((16, 4), full),
            pl.BlockSpec((16, 1), full),
            pl.BlockSpec((24, 256), full),
            pl.BlockSpec((24, 16), full),
        ],
        out_specs=pl.BlockSpec((32, eblk), lambda i: (0, i)),
        out_shape=jax.ShapeDtypeStruct((32, E_PAD), f32),
    )(eaT, xT, WeT, beT, W2T, B2T)

    zeros_init = jnp.zeros((N_NODES, 32), f32)
    partials = _sc_scatter(dst_idx, msgT, zeros_init)

    y = pl.pallas_call(
        _final_body,
        grid=(N_NODES // nblk,),
        in_specs=[
            pl.BlockSpec((NC, nblk, 32), lambda i: (0, i, 0)),
            pl.BlockSpec((nblk, 24), lambda i: (i, 0)),
            pl.BlockSpec((24, 2), full),
            pl.BlockSpec((1, 2), full),
        ],
        out_specs=pl.BlockSpec((nblk, 2), lambda i: (i, 0)),
        out_shape=jax.ShapeDtypeStruct((N_NODES, 2), f32),
    )(partials, root, W_out, b_out.reshape(1, 2))
    return y


# speedup vs baseline: 1.1975x; 1.1975x over previous
import functools

import jax
import jax.numpy as jnp
from jax import lax
from jax.experimental import pallas as pl
from jax.experimental.pallas import tpu as pltpu
from jax.experimental.pallas import tpu_sc as plsc

N_NODES = 10000
N_EDGES = 160000
E_PAD = 163840
NC = 2
NS = 16
NW = NC * NS
EPT = E_PAD // NW
CHUNK = 128
NCHUNK = EPT // CHUNK
STAGE = 1024
NSTAGE = EPT // STAGE
ROWS_PER_TILE = N_NODES // NS

_mesh = plsc.VectorSubcoreMesh(core_axis_name="c", subcore_axis_name="s")


def _prep_body(g0, mask, Wp, bp, Wt, bt, Wr, br, x_out, root_out):
    prep = jnp.dot(g0[...], Wp[...], preferred_element_type=jnp.float32) + bp[...]
    x0 = jnp.maximum(jnp.where(mask[...] == 0, prep, 0.0), 0.0)
    x = jnp.maximum(
        jnp.dot(x0, Wt[...], preferred_element_type=jnp.float32) + bt[...], 0.0)
    x_out[...] = x
    root_out[...] = (
        jnp.dot(x, Wr[...], preferred_element_type=jnp.float32) + br[...])


def _edge_body(eaT, xT, WeT, beT, W2T, B2T, out):
    hT = jax.nn.sigmoid(
        jnp.dot(WeT[...], eaT[...], preferred_element_type=jnp.float32)
        + beT[...])
    x = xT[...]
    b = x.shape[1]
    zT = (jnp.broadcast_to(hT[:, None, :], (16, 16, b)).reshape(256, b)
          * jnp.broadcast_to(x[None, :, :], (16, 16, b)).reshape(256, b))
    msgT = (jnp.dot(W2T[...], zT, preferred_element_type=jnp.float32)
            + jnp.dot(B2T[...], x, preferred_element_type=jnp.float32))
    pad = jnp.concatenate(
        [jnp.ones((1, b), jnp.float32), jnp.zeros((7, b), jnp.float32)], axis=0)
    msg32 = jnp.concatenate([msgT, pad], axis=0)
    eid = (jax.lax.broadcasted_iota(jnp.int32, (1, b), 1)
           + pl.program_id(0) * b)
    live = (eid < N_EDGES).astype(jnp.float32)
    out[...] = msg32 * live


def _final_body(p, root, Wo, bo, y_out):
    s = p[0] + p[1]
    cnt = jnp.maximum(s[:, 24:25], 1.0)
    mean_m = s[:, :16] / cnt
    h = jnp.concatenate(
        [mean_m + root[:, :16], s[:, 16:24] + root[:, 16:24]], axis=1)
    h = jnp.maximum(h, 0.0)
    y_out[...] = jnp.dot(h, Wo[...], preferred_element_type=jnp.float32) + bo[...]


@functools.partial(
    pl.kernel,
    mesh=_mesh,
    out_type=jax.ShapeDtypeStruct((16, E_PAD), jnp.float32),
    scratch_types=[
        pltpu.VMEM((NCHUNK, CHUNK), jnp.int32),
        pltpu.VMEM((STAGE, 16), jnp.float32),
        pltpu.VMEM((16, EPT), jnp.float32),
        pltpu.SemaphoreType.DMA,
    ],
    compiler_params=pltpu.CompilerParams(use_tc_tiling_on_sc=False,
                                         needs_layout_passes=False),
)
def _sc_gather(src_idx_hbm, x_hbm, out_hbm, idx_v, rows_v, xT_v, sem):
    c = lax.axis_index("c")
    s = lax.axis_index("s")
    wid = c * NS + s
    pltpu.sync_copy(src_idx_hbm.at[wid], idx_v)
    lane = lax.iota(jnp.int32, 16)

    def stage_body(k, carry):
        def fetch(j, carry2):
            pltpu.async_copy(
                x_hbm.at[idx_v.at[k * (STAGE // CHUNK) + j]],
                rows_v.at[pl.ds(j * CHUNK, CHUNK)], sem).wait()
            return carry2

        lax.fori_loop(0, STAGE // CHUNK, fetch, 0)

        @plsc.parallel_loop(0, STAGE, 1, unroll=8)
        def _transpose(e):
            v = rows_v[e, :]
            plsc.store_scatter(
                xT_v, [lane, jnp.zeros((16,), jnp.int32) + (k * STAGE + e)], v)

        return carry

    lax.fori_loop(0, NSTAGE, stage_body, 0)
    pltpu.sync_copy(xT_v, out_hbm.at[:, pl.ds(wid * EPT, EPT)])


@functools.partial(
    pl.kernel,
    mesh=_mesh,
    out_type=jax.ShapeDtypeStruct((NC, N_NODES, 32), jnp.float32),
    scratch_types=[
        pltpu.VMEM((NCHUNK, CHUNK), jnp.int32),
        pltpu.VMEM((32, STAGE), jnp.float32),
        pltpu.VMEM((STAGE, 32), jnp.float32),
        pltpu.VMEM_SHARED((N_NODES, 32), jnp.float32),
    ],
    compiler_params=pltpu.CompilerParams(use_tc_tiling_on_sc=False,
                                         needs_layout_passes=False),
)
def _sc_scatter(dst_idx_hbm, msgT_hbm, zeros_hbm, out_hbm,
                idx_v, slab_v, msg_v, accum):
    c = lax.axis_index("c")
    s = lax.axis_index("s")
    wid = c * NS + s
    pltpu.sync_copy(zeros_hbm.at[pl.ds(s * ROWS_PER_TILE, ROWS_PER_TILE)],
                    accum.at[pl.ds(s * ROWS_PER_TILE, ROWS_PER_TILE)])
    pltpu.sync_copy(dst_idx_hbm.at[wid], idx_v)
    plsc.subcore_barrier()

    base = wid * EPT
    lane = lax.iota(jnp.int32, 16)

    def stage_body(k, carry):
        pltpu.sync_copy(msgT_hbm.at[:, pl.ds(base + k * STAGE, STAGE)], slab_v)

        @plsc.parallel_loop(0, STAGE, 1, unroll=8)
        def _transpose(e):
            col = jnp.zeros((16,), jnp.int32) + e
            msg_v[e, pl.ds(0, 16)] = plsc.load_gather(slab_v, [lane, col])
            msg_v[e, pl.ds(16, 16)] = plsc.load_gather(slab_v, [16 + lane, col])

        def scat(j, carry3):
            pltpu.sync_copy(msg_v.at[pl.ds(j * CHUNK, CHUNK)],
                            accum.at[idx_v.at[k * (STAGE // CHUNK) + j]],
                            add=True)
            return carry3

        lax.fori_loop(0, STAGE // CHUNK, scat, 0)
        return carry

    lax.fori_loop(0, NSTAGE, stage_body, 0)
    plsc.subcore_barrier()
    pltpu.sync_copy(accum.at[pl.ds(s * ROWS_PER_TILE, ROWS_PER_TILE)],
                    out_hbm.at[c, pl.ds(s * ROWS_PER_TILE, ROWS_PER_TILE)])


def kernel(group0, edge_index, group_mask, edge_attr,
           W_prep, b_prep, W_tube, b_tube,
           W_e1, b_e1,
           W_em, b_em, W_root_m, bias_m,
           W_ea, b_ea, W_root_a, bias_a,
           W_out, b_out):
    f32 = jnp.float32
    mask2d = group_mask.astype(jnp.int32).reshape(N_NODES, 1)
    epad = jnp.zeros((E_PAD - N_EDGES,), jnp.int32)
    src_idx = jnp.concatenate(
        [edge_index[0].astype(jnp.int32), epad]).reshape(NW, NCHUNK, CHUNK)
    dst_idx = jnp.concatenate(
        [edge_index[1].astype(jnp.int32), epad]).reshape(NW, NCHUNK, CHUNK)

    Wr = jnp.concatenate([W_root_m, W_root_a], axis=1)
    br = jnp.concatenate([bias_m, bias_a]).reshape(1, 24)
    W2 = jnp.concatenate([W_em.reshape(16, 16, 16).reshape(256, 16),
                          W_ea.reshape(16, 16, 8).reshape(256, 8)], axis=1)
    B2 = jnp.concatenate([b_em.reshape(16, 16), b_ea.reshape(16, 8)], axis=1)
    W2T = W2.T
    B2T = B2.T
    WeT = W_e1.T
    beT = b_e1.reshape(16, 1)
    eaT = jnp.pad(edge_attr.T, ((0, 0), (0, E_PAD - N_EDGES)))

    nblk = 2000
    full = lambda i: (0, 0)
    x, root = pl.pallas_call(
        _prep_body,
        grid=(N_NODES // nblk,),
        in_specs=[
            pl.BlockSpec((nblk, 128), lambda i: (i, 0)),
            pl.BlockSpec((nblk, 1), lambda i: (i, 0)),
            pl.BlockSpec((128, 64), full),
            pl.BlockSpec((1, 64), full),
            pl.BlockSpec((64, 16), full),
            pl.BlockSpec((1, 16), full),
            pl.BlockSpec((16, 24), full),
            pl.BlockSpec((1, 24), full),
        ],
        out_specs=[
            pl.BlockSpec((nblk, 16), lambda i: (i, 0)),
            pl.BlockSpec((nblk, 24), lambda i: (i, 0)),
        ],
        out_shape=[
            jax.ShapeDtypeStruct((N_NODES, 16), f32),
            jax.ShapeDtypeStruct((N_NODES, 24), f32),
        ],
    )(group0, mask2d, W_prep, b_prep.reshape(1, 64),
      W_tube, b_tube.reshape(1, 16), Wr, br)

    xT = _sc_gather(src_idx, x)

    eblk = 4096
    msgT = pl.pallas_call(
        _edge_body,
        grid=(E_PAD // eblk,),
        in_specs=[
            pl.BlockSpec((4, eblk), lambda i: (0, i)),
            pl.BlockSpec((16, eblk), lambda i: (0, i)),
            pl.BlockSpec((16, 4), full),
            pl.BlockSpec((16, 1), full),
            pl.BlockSpec((24, 256), full),
            pl.BlockSpec((24, 16), full),
        ],
        out_specs=pl.BlockSpec((32, eblk), lambda i: (0, i)),
        out_shape=jax.ShapeDtypeStruct((32, E_PAD), f32),
    )(eaT, xT, WeT, beT, W2T, B2T)

    zeros_init = jnp.zeros((N_NODES, 32), f32)
    partials = _sc_scatter(dst_idx, msgT, zeros_init)

    y = pl.pallas_call(
        _final_body,
        grid=(N_NODES // nblk,),
        in_specs=[
            pl.BlockSpec((NC, nblk, 32), lambda i: (0, i, 0)),
            pl.BlockSpec((nblk, 24), lambda i: (i, 0)),
            pl.BlockSpec((24, 2), full),
            pl.BlockSpec((1, 2), full),
        ],
        out_specs=pl.BlockSpec((nblk, 2), lambda i: (i, 0)),
        out_shape=jax.ShapeDtypeStruct((N_NODES, 2), f32),
    )(partials, root, W_out, b_out.reshape(1, 2))
    return y


# speedup vs baseline: 1.2642x; 1.0556x over previous
import functools

import jax
import jax.numpy as jnp
from jax import lax
from jax.experimental import pallas as pl
from jax.experimental.pallas import tpu as pltpu
from jax.experimental.pallas import tpu_sc as plsc

N_NODES = 10000
N_EDGES = 160000
E_PAD = 163840
NC = 2
NS = 16
NW = NC * NS
EPT = E_PAD // NW
CHUNK = 128
NCHUNK = EPT // CHUNK
STAGE = 1024
NSTAGE = EPT // STAGE
ROWS_PER_TILE = N_NODES // NS

_mesh = plsc.VectorSubcoreMesh(core_axis_name="c", subcore_axis_name="s")


def _prep_body(g0, mask, Wp, bp, Wt, bt, Wr, br, x_out, root_out):
    prep = jnp.dot(g0[...], Wp[...], preferred_element_type=jnp.float32) + bp[...]
    x0 = jnp.maximum(jnp.where(mask[...] == 0, prep, 0.0), 0.0)
    x = jnp.maximum(
        jnp.dot(x0, Wt[...], preferred_element_type=jnp.float32) + bt[...], 0.0)
    x_out[...] = x
    root_out[...] = (
        jnp.dot(x, Wr[...], preferred_element_type=jnp.float32) + br[...])


def _edge_body(eaT, xT, WeT, beT, W2T, B2T, out):
    hT = jax.nn.sigmoid(
        jnp.dot(WeT[...], eaT[...], preferred_element_type=jnp.float32)
        + beT[...])
    x = xT[...]
    b = x.shape[1]
    zT = (jnp.broadcast_to(hT[:, None, :], (16, 16, b)).reshape(256, b)
          * jnp.broadcast_to(x[None, :, :], (16, 16, b)).reshape(256, b))
    msgT = (jnp.dot(W2T[...], zT, preferred_element_type=jnp.float32)
            + jnp.dot(B2T[...], x, preferred_element_type=jnp.float32))
    pad = jnp.concatenate(
        [jnp.ones((1, b), jnp.float32), jnp.zeros((7, b), jnp.float32)], axis=0)
    msg32 = jnp.concatenate([msgT, pad], axis=0)
    eid = (jax.lax.broadcasted_iota(jnp.int32, (1, b), 1)
           + pl.program_id(0) * b)
    live = (eid < N_EDGES).astype(jnp.float32)
    out[...] = msg32 * live


def _final_body(p, root, Wo, bo, y_out):
    s = p[0] + p[1]
    cnt = jnp.maximum(s[:, 24:25], 1.0)
    mean_m = s[:, :16] / cnt
    h = jnp.concatenate(
        [mean_m + root[:, :16], s[:, 16:24] + root[:, 16:24]], axis=1)
    h = jnp.maximum(h, 0.0)
    y_out[...] = jnp.dot(h, Wo[...], preferred_element_type=jnp.float32) + bo[...]


@functools.partial(
    pl.kernel,
    mesh=_mesh,
    out_type=jax.ShapeDtypeStruct((16, E_PAD), jnp.float32),
    scratch_types=[
        pltpu.VMEM((NCHUNK, CHUNK), jnp.int32),
        pltpu.VMEM((STAGE, 16), jnp.float32),
        pltpu.VMEM((16, EPT), jnp.float32),
        pltpu.SemaphoreType.DMA,
    ],
    compiler_params=pltpu.CompilerParams(use_tc_tiling_on_sc=False,
                                         needs_layout_passes=False),
)
def _sc_gather(src_idx_hbm, x_hbm, out_hbm, idx_v, rows_v, xT_v, sem):
    c = lax.axis_index("c")
    s = lax.axis_index("s")
    wid = c * NS + s
    pltpu.sync_copy(src_idx_hbm.at[wid], idx_v)
    lane = lax.iota(jnp.int32, 16)

    def stage_body(k, carry):
        def fire(j, carry2):
            pltpu.async_copy(
                x_hbm.at[idx_v.at[k * (STAGE // CHUNK) + j]],
                rows_v.at[pl.ds(j * CHUNK, CHUNK)], sem)
            return carry2

        lax.fori_loop(0, STAGE // CHUNK, fire, 0)

        def drain(j, carry2):
            pltpu.make_async_copy(
                x_hbm.at[idx_v.at[k * (STAGE // CHUNK) + j]],
                rows_v.at[pl.ds(j * CHUNK, CHUNK)], sem).wait()
            return carry2

        lax.fori_loop(0, STAGE // CHUNK, drain, 0)

        @plsc.parallel_loop(0, STAGE, 1, unroll=16)
        def _transpose(e):
            v = rows_v[e, :]
            plsc.store_scatter(
                xT_v, [lane, jnp.zeros((16,), jnp.int32) + (k * STAGE + e)], v)

        return carry

    lax.fori_loop(0, NSTAGE, stage_body, 0)
    pltpu.sync_copy(xT_v, out_hbm.at[:, pl.ds(wid * EPT, EPT)])


@functools.partial(
    pl.kernel,
    mesh=_mesh,
    out_type=jax.ShapeDtypeStruct((NC, N_NODES, 32), jnp.float32),
    scratch_types=[
        pltpu.VMEM((NCHUNK, CHUNK), jnp.int32),
        pltpu.VMEM((32, STAGE), jnp.float32),
        pltpu.VMEM((STAGE, 32), jnp.float32),
        pltpu.VMEM((ROWS_PER_TILE, 32), jnp.float32),
        pltpu.VMEM_SHARED((N_NODES, 32), jnp.float32),
        pltpu.SemaphoreType.DMA,
    ],
    compiler_params=pltpu.CompilerParams(use_tc_tiling_on_sc=False,
                                         needs_layout_passes=False),
)
def _sc_scatter(dst_idx_hbm, msgT_hbm, out_hbm,
                idx_v, slab_v, msg_v, zbuf, accum, sem):
    c = lax.axis_index("c")
    s = lax.axis_index("s")
    wid = c * NS + s

    @plsc.parallel_loop(0, ROWS_PER_TILE, 1, unroll=8)
    def _zero(r):
        zv = jnp.zeros((16,), jnp.float32)
        zbuf[r, pl.ds(0, 16)] = zv
        zbuf[r, pl.ds(16, 16)] = zv

    pltpu.sync_copy(zbuf, accum.at[pl.ds(s * ROWS_PER_TILE, ROWS_PER_TILE)])
    pltpu.sync_copy(dst_idx_hbm.at[wid], idx_v)
    plsc.subcore_barrier()

    base = wid * EPT
    lane = lax.iota(jnp.int32, 16)

    def stage_body(k, carry):
        pltpu.sync_copy(msgT_hbm.at[:, pl.ds(base + k * STAGE, STAGE)], slab_v)

        @plsc.parallel_loop(0, STAGE, 1, unroll=16)
        def _transpose(e):
            col = jnp.zeros((16,), jnp.int32) + e
            msg_v[e, pl.ds(0, 16)] = plsc.load_gather(slab_v, [lane, col])
            msg_v[e, pl.ds(16, 16)] = plsc.load_gather(slab_v, [16 + lane, col])

        def fire_s(j, carry3):
            pltpu.async_copy(msg_v.at[pl.ds(j * CHUNK, CHUNK)],
                             accum.at[idx_v.at[k * (STAGE // CHUNK) + j]],
                             sem, add=True)
            return carry3

        lax.fori_loop(0, STAGE // CHUNK, fire_s, 0)

        def drain_s(j, carry3):
            pltpu.make_async_copy(
                msg_v.at[pl.ds(j * CHUNK, CHUNK)],
                accum.at[idx_v.at[k * (STAGE // CHUNK) + j]], sem).wait()
            return carry3

        lax.fori_loop(0, STAGE // CHUNK, drain_s, 0)
        return carry

    lax.fori_loop(0, NSTAGE, stage_body, 0)
    plsc.subcore_barrier()
    pltpu.sync_copy(accum.at[pl.ds(s * ROWS_PER_TILE, ROWS_PER_TILE)],
                    out_hbm.at[c, pl.ds(s * ROWS_PER_TILE, ROWS_PER_TILE)])


def kernel(group0, edge_index, group_mask, edge_attr,
           W_prep, b_prep, W_tube, b_tube,
           W_e1, b_e1,
           W_em, b_em, W_root_m, bias_m,
           W_ea, b_ea, W_root_a, bias_a,
           W_out, b_out):
    f32 = jnp.float32
    mask2d = group_mask.astype(jnp.int32).reshape(N_NODES, 1)
    epad = jnp.zeros((E_PAD - N_EDGES,), jnp.int32)
    src_idx = jnp.concatenate(
        [edge_index[0].astype(jnp.int32), epad]).reshape(NW, NCHUNK, CHUNK)
    dst_idx = jnp.concatenate(
        [edge_index[1].astype(jnp.int32), epad]).reshape(NW, NCHUNK, CHUNK)

    Wr = jnp.concatenate([W_root_m, W_root_a], axis=1)
    br = jnp.concatenate([bias_m, bias_a]).reshape(1, 24)
    W2 = jnp.concatenate([W_em.reshape(16, 16, 16).reshape(256, 16),
                          W_ea.reshape(16, 16, 8).reshape(256, 8)], axis=1)
    B2 = jnp.concatenate([b_em.reshape(16, 16), b_ea.reshape(16, 8)], axis=1)
    W2T = W2.T
    B2T = B2.T
    WeT = W_e1.T
    beT = b_e1.reshape(16, 1)
    eaT = jnp.pad(edge_attr.T, ((0, 0), (0, E_PAD - N_EDGES)))

    nblk = 2000
    full = lambda i: (0, 0)
    x, root = pl.pallas_call(
        _prep_body,
        grid=(N_NODES // nblk,),
        in_specs=[
            pl.BlockSpec((nblk, 128), lambda i: (i, 0)),
            pl.BlockSpec((nblk, 1), lambda i: (i, 0)),
            pl.BlockSpec((128, 64), full),
            pl.BlockSpec((1, 64), full),
            pl.BlockSpec((64, 16), full),
            pl.BlockSpec((1, 16), full),
            pl.BlockSpec((16, 24), full),
            pl.BlockSpec((1, 24), full),
        ],
        out_specs=[
            pl.BlockSpec((nblk, 16), lambda i: (i, 0)),
            pl.BlockSpec((nblk, 24), lambda i: (i, 0)),
        ],
        out_shape=[
            jax.ShapeDtypeStruct((N_NODES, 16), f32),
            jax.ShapeDtypeStruct((N_NODES, 24), f32),
        ],
    )(group0, mask2d, W_prep, b_prep.reshape(1, 64),
      W_tube, b_tube.reshape(1, 16), Wr, br)

    xT = _sc_gather(src_idx, x)

    eblk = 4096
    msgT = pl.pallas_call(
        _edge_body,
        grid=(E_PAD // eblk,),
        in_specs=[
            pl.BlockSpec((4, eblk), lambda i: (0, i)),
            pl.BlockSpec((16, eblk), lambda i: (0, i)),
            pl.BlockSpec((16, 4), full),
            pl.BlockSpec((16, 1), full),
            pl.BlockSpec((24, 256), full),
            pl.BlockSpec((24, 16), full),
        ],
        out_specs=pl.BlockSpec((32, eblk), lambda i: (0, i)),
        out_shape=jax.ShapeDtypeStruct((32, E_PAD), f32),
    )(eaT, xT, WeT, beT, W2T, B2T)

    partials = _sc_scatter(dst_idx, msgT)

    y = pl.pallas_call(
        _final_body,
        grid=(N_NODES // nblk,),
        in_specs=[
            pl.BlockSpec((NC, nblk, 32), lambda i: (0, i, 0)),
            pl.BlockSpec((nblk, 24), lambda i: (i, 0)),
            pl.BlockSpec((24, 2), full),
            pl.BlockSpec((1, 2), full),
        ],
        out_specs=pl.BlockSpec((nblk, 2), lambda i: (i, 0)),
        out_shape=jax.ShapeDtypeStruct((N_NODES, 2), f32),
    )(partials, root, W_out, b_out.reshape(1, 2))
    return y


# speedup vs baseline: 1.3322x; 1.0538x over previous
import functools

import jax
import jax.numpy as jnp
from jax import lax
from jax.experimental import pallas as pl
from jax.experimental.pallas import tpu as pltpu
from jax.experimental.pallas import tpu_sc as plsc

N_NODES = 10000
N_EDGES = 160000
E_HALF = 81920
E_PAD = 2 * E_HALF
NC = 2
NS = 16
NW = NC * NS
EPT = E_HALF // NW
CHUNK = 128
NCHUNK = EPT // CHUNK
STAGE = 1280
NSTAGE = EPT // STAGE
CPS = STAGE // CHUNK
ROWS_PER_TILE = N_NODES // NS

_mesh = plsc.VectorSubcoreMesh(core_axis_name="c", subcore_axis_name="s")


def _prep_body(g0, mask, Wp, bp, Wt, bt, Wr, br, x_out, root_out):
    prep = jnp.dot(g0[...], Wp[...], preferred_element_type=jnp.float32) + bp[...]
    x0 = jnp.maximum(jnp.where(mask[...] == 0, prep, 0.0), 0.0)
    x = jnp.maximum(
        jnp.dot(x0, Wt[...], preferred_element_type=jnp.float32) + bt[...], 0.0)
    x_out[...] = x
    root_out[...] = (
        jnp.dot(x, Wr[...], preferred_element_type=jnp.float32) + br[...])


def _make_edge_body(base):
    def _edge_body(eaT, xT, WeT, beT, W2T, B2T, out):
        hT = jax.nn.sigmoid(
            jnp.dot(WeT[...], eaT[...], preferred_element_type=jnp.float32)
            + beT[...])
        x = xT[...]
        b = x.shape[1]
        zT = (jnp.broadcast_to(hT[:, None, :], (16, 16, b)).reshape(256, b)
              * jnp.broadcast_to(x[None, :, :], (16, 16, b)).reshape(256, b))
        msgT = (jnp.dot(W2T[...], zT, preferred_element_type=jnp.float32)
                + jnp.dot(B2T[...], x, preferred_element_type=jnp.float32))
        pad = jnp.concatenate(
            [jnp.ones((1, b), jnp.float32), jnp.zeros((7, b), jnp.float32)],
            axis=0)
        msg32 = jnp.concatenate([msgT, pad], axis=0)
        eid = (jax.lax.broadcasted_iota(jnp.int32, (1, b), 1)
               + pl.program_id(0) * b + base)
        live = (eid < N_EDGES).astype(jnp.float32)
        out[...] = msg32 * live
    return _edge_body


def _final_body(p1, p2, root, Wo, bo, y_out):
    s = p1[0] + p1[1] + p2[0] + p2[1]
    cnt = jnp.maximum(s[:, 24:25], 1.0)
    mean_m = s[:, :16] / cnt
    h = jnp.concatenate(
        [mean_m + root[:, :16], s[:, 16:24] + root[:, 16:24]], axis=1)
    h = jnp.maximum(h, 0.0)
    y_out[...] = jnp.dot(h, Wo[...], preferred_element_type=jnp.float32) + bo[...]


@functools.partial(
    pl.kernel,
    mesh=_mesh,
    out_type=jax.ShapeDtypeStruct((16, E_HALF), jnp.float32),
    scratch_types=[
        pltpu.VMEM((NCHUNK, CHUNK), jnp.int32),
        pltpu.VMEM((STAGE, 16), jnp.float32),
        pltpu.VMEM((16, EPT), jnp.float32),
        pltpu.SemaphoreType.DMA,
    ],
    compiler_params=pltpu.CompilerParams(use_tc_tiling_on_sc=False,
                                         needs_layout_passes=False),
)
def _sc_gather(src_idx_hbm, x_hbm, out_hbm, idx_v, rows_v, xT_v, sem):
    c = lax.axis_index("c")
    s = lax.axis_index("s")
    wid = c * NS + s
    pltpu.sync_copy(src_idx_hbm.at[wid], idx_v)
    lane = lax.iota(jnp.int32, 16)

    def stage_body(k, carry):
        def fire(j, carry2):
            pltpu.async_copy(
                x_hbm.at[idx_v.at[k * CPS + j]],
                rows_v.at[pl.ds(j * CHUNK, CHUNK)], sem)
            return carry2

        lax.fori_loop(0, CPS, fire, 0)

        def drain(j, carry2):
            pltpu.make_async_copy(
                x_hbm.at[idx_v.at[k * CPS + j]],
                rows_v.at[pl.ds(j * CHUNK, CHUNK)], sem).wait()
            return carry2

        lax.fori_loop(0, CPS, drain, 0)

        @plsc.parallel_loop(0, STAGE, 1, unroll=16)
        def _transpose(e):
            v = rows_v[e, :]
            plsc.store_scatter(
                xT_v, [lane, jnp.zeros((16,), jnp.int32) + (k * STAGE + e)], v)

        return carry

    lax.fori_loop(0, NSTAGE, stage_body, 0)
    pltpu.sync_copy(xT_v, out_hbm.at[:, pl.ds(wid * EPT, EPT)])


@functools.partial(
    pl.kernel,
    mesh=_mesh,
    out_type=jax.ShapeDtypeStruct((NC, N_NODES, 32), jnp.float32),
    scratch_types=[
        pltpu.VMEM((NCHUNK, CHUNK), jnp.int32),
        pltpu.VMEM((32, STAGE), jnp.float32),
        pltpu.VMEM((STAGE, 32), jnp.float32),
        pltpu.VMEM((ROWS_PER_TILE, 32), jnp.float32),
        pltpu.VMEM_SHARED((N_NODES, 32), jnp.float32),
        pltpu.SemaphoreType.DMA,
    ],
    compiler_params=pltpu.CompilerParams(use_tc_tiling_on_sc=False,
                                         needs_layout_passes=False),
)
def _sc_scatter(dst_idx_hbm, msgT_hbm, out_hbm,
                idx_v, slab_v, msg_v, zbuf, accum, sem):
    c = lax.axis_index("c")
    s = lax.axis_index("s")
    wid = c * NS + s

    @plsc.parallel_loop(0, ROWS_PER_TILE, 1, unroll=8)
    def _zero(r):
        zv = jnp.zeros((16,), jnp.float32)
        zbuf[r, pl.ds(0, 16)] = zv
        zbuf[r, pl.ds(16, 16)] = zv

    pltpu.sync_copy(zbuf, accum.at[pl.ds(s * ROWS_PER_TILE, ROWS_PER_TILE)])
    pltpu.sync_copy(dst_idx_hbm.at[wid], idx_v)
    plsc.subcore_barrier()

    base = wid * EPT
    lane = lax.iota(jnp.int32, 16)

    def stage_body(k, carry):
        pltpu.sync_copy(msgT_hbm.at[:, pl.ds(base + k * STAGE, STAGE)], slab_v)

        @plsc.parallel_loop(0, STAGE, 1, unroll=16)
        def _transpose(e):
            col = jnp.zeros((16,), jnp.int32) + e
            msg_v[e, pl.ds(0, 16)] = plsc.load_gather(slab_v, [lane, col])
            msg_v[e, pl.ds(16, 16)] = plsc.load_gather(slab_v, [16 + lane, col])

        def fire_s(j, carry3):
            pltpu.async_copy(msg_v.at[pl.ds(j * CHUNK, CHUNK)],
                             accum.at[idx_v.at[k * CPS + j]],
                             sem, add=True)
            return carry3

        lax.fori_loop(0, CPS, fire_s, 0)

        def drain_s(j, carry3):
            pltpu.make_async_copy(
                msg_v.at[pl.ds(j * CHUNK, CHUNK)],
                accum.at[idx_v.at[k * CPS + j]], sem).wait()
            return carry3

        lax.fori_loop(0, CPS, drain_s, 0)
        return carry

    lax.fori_loop(0, NSTAGE, stage_body, 0)
    plsc.subcore_barrier()
    pltpu.sync_copy(accum.at[pl.ds(s * ROWS_PER_TILE, ROWS_PER_TILE)],
                    out_hbm.at[c, pl.ds(s * ROWS_PER_TILE, ROWS_PER_TILE)])


def _edge_call(eaT_h, xT_h, base, WeT, beT, W2T, B2T):
    eblk = 4096
    full = lambda i: (0, 0)
    return pl.pallas_call(
        _make_edge_body(base),
        grid=(E_HALF // eblk,),
        in_specs=[
            pl.BlockSpec((4, eblk), lambda i: (0, i)),
            pl.BlockSpec((16, eblk), lambda i: (0, i)),
            pl.BlockSpec((16, 4), full),
            pl.BlockSpec((16, 1), full),
            pl.BlockSpec((24, 256), full),
            pl.BlockSpec((24, 16), full),
        ],
        out_specs=pl.BlockSpec((32, eblk), lambda i: (0, i)),
        out_shape=jax.ShapeDtypeStruct((32, E_HALF), jnp.float32),
    )(eaT_h, xT_h, WeT, beT, W2T, B2T)


def kernel(group0, edge_index, group_mask, edge_attr,
           W_prep, b_prep, W_tube, b_tube,
           W_e1, b_e1,
           W_em, b_em, W_root_m, bias_m,
           W_ea, b_ea, W_root_a, bias_a,
           W_out, b_out):
    f32 = jnp.float32
    mask2d = group_mask.astype(jnp.int32).reshape(N_NODES, 1)
    epad = jnp.zeros((E_PAD - N_EDGES,), jnp.int32)
    src_idx = jnp.concatenate(
        [edge_index[0].astype(jnp.int32), epad]).reshape(2, NW, NCHUNK, CHUNK)
    dst_idx = jnp.concatenate(
        [edge_index[1].astype(jnp.int32), epad]).reshape(2, NW, NCHUNK, CHUNK)

    Wr = jnp.concatenate([W_root_m, W_root_a], axis=1)
    br = jnp.concatenate([bias_m, bias_a]).reshape(1, 24)
    W2 = jnp.concatenate([W_em.reshape(16, 16, 16).reshape(256, 16),
                          W_ea.reshape(16, 16, 8).reshape(256, 8)], axis=1)
    B2 = jnp.concatenate([b_em.reshape(16, 16), b_ea.reshape(16, 8)], axis=1)
    W2T = W2.T
    B2T = B2.T
    WeT = W_e1.T
    beT = b_e1.reshape(16, 1)
    eaT = jnp.pad(edge_attr.T, ((0, 0), (0, E_PAD - N_EDGES)))

    nblk = 2000
    full = lambda i: (0, 0)
    x, root = pl.pallas_call(
        _prep_body,
        grid=(N_NODES // nblk,),
        in_specs=[
            pl.BlockSpec((nblk, 128), lambda i: (i, 0)),
            pl.BlockSpec((nblk, 1), lambda i: (i, 0)),
            pl.BlockSpec((128, 64), full),
            pl.BlockSpec((1, 64), full),
            pl.BlockSpec((64, 16), full),
            pl.BlockSpec((1, 16), full),
            pl.BlockSpec((16, 24), full),
            pl.BlockSpec((1, 24), full),
        ],
        out_specs=[
            pl.BlockSpec((nblk, 16), lambda i: (i, 0)),
            pl.BlockSpec((nblk, 24), lambda i: (i, 0)),
        ],
        out_shape=[
            jax.ShapeDtypeStruct((N_NODES, 16), f32),
            jax.ShapeDtypeStruct((N_NODES, 24), f32),
        ],
    )(group0, mask2d, W_prep, b_prep.reshape(1, 64),
      W_tube, b_tube.reshape(1, 16), Wr, br)

    xT1 = _sc_gather(src_idx[0], x)
    xT2 = _sc_gather(src_idx[1], x)
    msgT1 = _edge_call(eaT[:, :E_HALF], xT1, 0, WeT, beT, W2T, B2T)
    partials1 = _sc_scatter(dst_idx[0], msgT1)
    msgT2 = _edge_call(eaT[:, E_HALF:], xT2, E_HALF, WeT, beT, W2T, B2T)
    partials2 = _sc_scatter(dst_idx[1], msgT2)

    y = pl.pallas_call(
        _final_body,
        grid=(N_NODES // nblk,),
        in_specs=[
            pl.BlockSpec((NC, nblk, 32), lambda i: (0, i, 0)),
            pl.BlockSpec((NC, nblk, 32), lambda i: (0, i, 0)),
            pl.BlockSpec((nblk, 24), lambda i: (i, 0)),
            pl.BlockSpec((24, 2), full),
            pl.BlockSpec((1, 2), full),
        ],
        out_specs=pl.BlockSpec((nblk, 2), lambda i: (i, 0)),
        out_shape=jax.ShapeDtypeStruct((N_NODES, 2), f32),
    )(partials1, partials2, root, W_out, b_out.reshape(1, 2))
    return y
